# baseline (speedup 1.0000x reference)
import functools

import jax
import jax.numpy as jnp
from jax.experimental import pallas as pl
from jax.experimental.pallas import tpu as pltpu

_ALPHA = 1.0
_GAMMA = 2.0
_SMOOTH = 1e-07

_B = 1024
_S = 200
_BR = 8


def _pair_loss(d):
    ad = jnp.abs(d)
    e = jnp.exp(-ad)
    sp = jnp.maximum(-d, 0.0) + jnp.log1p(e)
    recip = 1.0 / (1.0 + e)
    pt = jnp.where(d >= 0, recip, e * recip)
    pt = jnp.clip(pt, _SMOOTH, 1.0 - _SMOOTH)
    om = 1.0 - pt
    return _ALPHA * om * om * sp


def _dense_body(s_ref, st_ref, t_ref, tl_ref, tt_ref, tlt_ref, out_ref):
    @pl.when(pl.program_id(0) == 0)
    def _():
        out_ref[0, 0] = 0.0

    acc = 0.0
    for r in range(_BR):
        n_row = s_ref[r : r + 1, :]
        p_col = st_ref[0, :, r : r + 1]
        posm = (tt_ref[0, :, r : r + 1] >= 1) & (tlt_ref[0, :, r : r + 1] != 0)
        negm = (t_ref[r : r + 1, :] == 0) & (tl_ref[r : r + 1, :] != 0)
        d = p_col - n_row
        f = _pair_loss(d)
        m = posm & negm
        row_sum = jnp.sum(jnp.where(m, f, 0.0))
        pcnt = jnp.sum(posm.astype(jnp.float32))
        ncnt = jnp.sum(negm.astype(jnp.float32))
        cnt = pcnt * ncnt
        acc += jnp.where(cnt > 0, row_sum / jnp.maximum(cnt, 1.0), 0.0)
    out_ref[0, 0] += acc


@jax.jit
def kernel(scores, targets, target_len):
    t = targets.astype(jnp.int32)
    tl = target_len.astype(jnp.int32)
    grid = (_B // _BR,)
    out = pl.pallas_call(
        _dense_body,
        grid=grid,
        in_specs=[
            pl.BlockSpec((_BR, _S), lambda i: (i, 0)),
            pl.BlockSpec((1, _S, _BR), lambda i: (i, 0, 0)),
            pl.BlockSpec((_BR, _S), lambda i: (i, 0)),
            pl.BlockSpec((_BR, _S), lambda i: (i, 0)),
            pl.BlockSpec((1, _S, _BR), lambda i: (i, 0, 0)),
            pl.BlockSpec((1, _S, _BR), lambda i: (i, 0, 0)),
        ],
        out_specs=pl.BlockSpec(memory_space=pltpu.SMEM),
        out_shape=jax.ShapeDtypeStruct((1, 1), jnp.float32),
    )(
        scores,
        scores.reshape(_B // _BR, _BR, _S).transpose(0, 2, 1),
        t,
        tl,
        t.reshape(_B // _BR, _BR, _S).transpose(0, 2, 1),
        tl.reshape(_B // _BR, _BR, _S).transpose(0, 2, 1),
    )
    return out[0, 0] / _B
